# baseline (device time: 18284 ns/iter reference)
import jax
import jax.numpy as jnp
from jax import lax
from jax.experimental import pallas as pl
from jax.experimental.pallas import tpu as pltpu

N_DEV = 4
TAPS = 4
HALO = TAPS - 1
CHUNK = 256
PADH = 8
OFF = PADH - HALO


def kernel(x, k):
    b, s, c = x.shape
    n_chunks = s // CHUNK

    def body(x_hbm, k_ref, out_hbm, xbuf, obuf, halo_ref,
             in_sems, out_sems, send_sem, recv_sem):
        my = lax.axis_index("i")
        left = (my - 1) % N_DEV
        right = (my + 1) % N_DEV

        barrier_sem = pltpu.get_barrier_semaphore()
        for nbr in [left, right]:
            pl.semaphore_signal(
                barrier_sem, inc=1,
                device_id=(nbr,), device_id_type=pl.DeviceIdType.MESH,
            )
        pl.semaphore_wait(barrier_sem, 2)

        rdma = pltpu.make_async_remote_copy(
            src_ref=x_hbm.at[:, pl.ds(s - HALO, HALO), :],
            dst_ref=halo_ref,
            send_sem=send_sem,
            recv_sem=recv_sem,
            device_id=(right,),
            device_id_type=pl.DeviceIdType.MESH,
        )
        rdma.start()

        def in_copy(j):
            slot = j % 2
            if j == 0:
                return pltpu.make_async_copy(
                    x_hbm.at[:, pl.ds(0, CHUNK), :],
                    xbuf.at[slot, :, pl.ds(PADH, CHUNK), :],
                    in_sems.at[slot],
                )
            return pltpu.make_async_copy(
                x_hbm.at[:, pl.ds(j * CHUNK - PADH, CHUNK + PADH), :],
                xbuf.at[slot],
                in_sems.at[slot],
            )

        def out_copy(j):
            slot = j % 2
            return pltpu.make_async_copy(
                obuf.at[slot],
                out_hbm.at[:, pl.ds(j * CHUNK, CHUNK), :],
                out_sems.at[slot],
            )

        in_copy(0).start()
        in_copy(1).start()

        kv = k_ref[...].astype(jnp.bfloat16)
        one = jnp.bfloat16(1.0)

        for j in range(n_chunks):
            slot = j % 2
            in_copy(j).wait()
            if j == 0:
                rdma.wait_recv()
                xbuf[0, :, OFF:PADH, :] = jnp.where(my == 0, 0.0, halo_ref[...])
                rdma.wait_send()
            xb = xbuf[slot].astype(jnp.bfloat16)
            acc = xb[:, OFF:OFF + CHUNK, :] * kv[0][None, None, :]
            for t in range(1, TAPS):
                acc += xb[:, OFF + t:OFF + t + CHUNK, :] * kv[t][None, None, :]
            if j >= 2:
                out_copy(j - 2).wait()
            obuf[slot] = acc * (one / (one + jnp.exp(-acc)))
            out_copy(j).start()
            if j + 2 < n_chunks:
                in_copy(j + 2).start()

        out_copy(n_chunks - 2).wait()
        out_copy(n_chunks - 1).wait()

    return pl.pallas_call(
        body,
        out_shape=jax.ShapeDtypeStruct((b, s, c), jnp.bfloat16),
        in_specs=[
            pl.BlockSpec(memory_space=pl.ANY),
            pl.BlockSpec(memory_space=pltpu.VMEM),
        ],
        out_specs=pl.BlockSpec(memory_space=pl.ANY),
        scratch_shapes=[
            pltpu.VMEM((2, b, CHUNK + PADH, c), x.dtype),
            pltpu.VMEM((2, b, CHUNK, c), jnp.bfloat16),
            pltpu.VMEM((b, HALO, c), x.dtype),
            pltpu.SemaphoreType.DMA((2,)),
            pltpu.SemaphoreType.DMA((2,)),
            pltpu.SemaphoreType.DMA,
            pltpu.SemaphoreType.DMA,
        ],
        compiler_params=pltpu.CompilerParams(collective_id=0),
    )(x, k)


# device time: 15412 ns/iter; 1.1863x vs baseline; 1.1863x over previous
import jax
import jax.numpy as jnp
from jax import lax
from jax.experimental import pallas as pl
from jax.experimental.pallas import tpu as pltpu

N_DEV = 4
TAPS = 4
HALO = TAPS - 1
CHUNK = 256
PADH = 8
OFF = PADH - HALO


def kernel(x, k):
    b, s, c = x.shape
    n_chunks = s // CHUNK

    def body(x_hbm, k_ref, out_hbm, xbuf, obuf, halo_ref,
             in_sems, out_sems, send_sem, recv_sem):
        my = lax.axis_index("i")
        left = (my - 1) % N_DEV
        right = (my + 1) % N_DEV

        barrier_sem = pltpu.get_barrier_semaphore()
        for nbr in [left, right]:
            pl.semaphore_signal(
                barrier_sem, inc=1,
                device_id=(nbr,), device_id_type=pl.DeviceIdType.MESH,
            )
        pl.semaphore_wait(barrier_sem, 2)

        rdma = pltpu.make_async_remote_copy(
            src_ref=x_hbm.at[:, pl.ds(s - HALO, HALO), :],
            dst_ref=halo_ref,
            send_sem=send_sem,
            recv_sem=recv_sem,
            device_id=(right,),
            device_id_type=pl.DeviceIdType.MESH,
        )
        rdma.start()

        def in_copy(j):
            if j == 0:
                return pltpu.make_async_copy(
                    x_hbm.at[:, pl.ds(0, CHUNK), :],
                    xbuf.at[j, :, pl.ds(PADH, CHUNK), :],
                    in_sems.at[j],
                )
            return pltpu.make_async_copy(
                x_hbm.at[:, pl.ds(j * CHUNK - PADH, CHUNK + PADH), :],
                xbuf.at[j],
                in_sems.at[j],
            )

        def out_copy(j):
            return pltpu.make_async_copy(
                obuf.at[j],
                out_hbm.at[:, pl.ds(j * CHUNK, CHUNK), :],
                out_sems.at[j],
            )

        order = [j % n_chunks for j in range(1, n_chunks + 1)]
        for j in order:
            in_copy(j).start()

        kv = k_ref[...].astype(jnp.bfloat16)
        one = jnp.bfloat16(1.0)

        for j in order:
            in_copy(j).wait()
            if j == 0:
                rdma.wait_recv()
                xbuf[0, :, OFF:PADH, :] = jnp.where(my == 0, 0.0, halo_ref[...])
                rdma.wait_send()
            xb = xbuf[j].astype(jnp.bfloat16)
            acc = xb[:, OFF:OFF + CHUNK, :] * kv[0][None, None, :]
            for t in range(1, TAPS):
                acc += xb[:, OFF + t:OFF + t + CHUNK, :] * kv[t][None, None, :]
            obuf[j] = acc * (one / (one + jnp.exp(-acc)))
            out_copy(j).start()

        for j in order:
            out_copy(j).wait()

    return pl.pallas_call(
        body,
        out_shape=jax.ShapeDtypeStruct((b, s, c), jnp.bfloat16),
        in_specs=[
            pl.BlockSpec(memory_space=pl.ANY),
            pl.BlockSpec(memory_space=pltpu.VMEM),
        ],
        out_specs=pl.BlockSpec(memory_space=pl.ANY),
        scratch_shapes=[
            pltpu.VMEM((n_chunks, b, CHUNK + PADH, c), x.dtype),
            pltpu.VMEM((n_chunks, b, CHUNK, c), jnp.bfloat16),
            pltpu.VMEM((b, HALO, c), x.dtype),
            pltpu.SemaphoreType.DMA((n_chunks,)),
            pltpu.SemaphoreType.DMA((n_chunks,)),
            pltpu.SemaphoreType.DMA,
            pltpu.SemaphoreType.DMA,
        ],
        compiler_params=pltpu.CompilerParams(collective_id=0),
    )(x, k)


# device time: 10085 ns/iter; 1.8130x vs baseline; 1.5282x over previous
import jax
import jax.numpy as jnp
from jax import lax
from jax.experimental import pallas as pl
from jax.experimental.pallas import tpu as pltpu

N_DEV = 4
TAPS = 4
HALO = TAPS - 1

DO_CONV = True
DO_SILU = True


def kernel(x, k):
    b, s, c = x.shape

    def body(x_ref, k_ref, out_ref, halo_ref, send_sem, recv_sem):
        my = lax.axis_index("i")
        left = (my - 1) % N_DEV
        right = (my + 1) % N_DEV

        barrier_sem = pltpu.get_barrier_semaphore()
        for nbr in [left, right]:
            pl.semaphore_signal(
                barrier_sem, inc=1,
                device_id=(nbr,), device_id_type=pl.DeviceIdType.MESH,
            )
        pl.semaphore_wait(barrier_sem, 2)

        rdma = pltpu.make_async_remote_copy(
            src_ref=x_ref.at[:, pl.ds(s - HALO, HALO), :],
            dst_ref=halo_ref,
            send_sem=send_sem,
            recv_sem=recv_sem,
            device_id=(right,),
            device_id_type=pl.DeviceIdType.MESH,
        )
        rdma.start()

        one = jnp.bfloat16(1.0)
        xv = x_ref[...].astype(jnp.bfloat16)
        kv = k_ref[...].astype(jnp.bfloat16)

        if DO_CONV:
            acc = xv * kv[TAPS - 1][None, None, :]
            for d in range(1, TAPS):
                acc += (
                    pltpu.roll(xv, d, axis=1)
                    * kv[TAPS - 1 - d][None, None, :]
                )
        else:
            acc = xv
        if DO_SILU:
            out_ref[...] = acc * (one / (one + jnp.exp(-acc)))
        else:
            out_ref[...] = acc

        rdma.wait()

        halo = jnp.where(my == 0, 0.0, halo_ref[...]).astype(jnp.bfloat16)
        head = jnp.concatenate([halo, xv[:, 0:HALO, :]], axis=1)
        acc_h = head[:, 0:HALO, :] * kv[0][None, None, :]
        for t in range(1, TAPS - 1):
            acc_h += head[:, t:t + HALO, :] * kv[t][None, None, :]
        acc_h += xv[:, 0:HALO, :] * kv[TAPS - 1][None, None, :]
        out_ref[:, 0:HALO, :] = acc_h * (one / (one + jnp.exp(-acc_h)))

    return pl.pallas_call(
        body,
        out_shape=jax.ShapeDtypeStruct((b, s, c), jnp.bfloat16),
        in_specs=[
            pl.BlockSpec(memory_space=pltpu.VMEM),
            pl.BlockSpec(memory_space=pltpu.VMEM),
        ],
        out_specs=pl.BlockSpec(memory_space=pltpu.VMEM),
        scratch_shapes=[
            pltpu.VMEM((b, HALO, c), x.dtype),
            pltpu.SemaphoreType.DMA,
            pltpu.SemaphoreType.DMA,
        ],
        compiler_params=pltpu.CompilerParams(collective_id=0),
    )(x, k)
